# baseline (speedup 1.0000x reference)
import functools

import jax
import jax.numpy as jnp
from jax.experimental import pallas as pl
from jax.experimental.pallas import tpu as pltpu
from jax.experimental.pallas import tpu_sc as plsc

_GAMMA = -0.5
_ZETA = 1.1


def _node_pre(x, gcn_W, gcn_b, nb_W, nb_b, self_W, self_b, attW1, attW2, att_b):
    n, d = x.shape
    h_dim = gcn_W.shape[1]
    blk = 2000 if n % 2000 == 0 else n

    def body(x_ref, gw, gb, nw, nbb, sw, sb, a1, a2, ab, h_ref, p_ref, q_ref):
        xb = x_ref[...]
        h_ref[...] = xb @ gw[...] + gb[...]
        f1 = jnp.maximum(xb @ nw[...] + nbb[...], 0.0)
        p_ref[...] = f1 @ a1[...] + ab[...]
        f2 = jnp.maximum(xb @ sw[...] + sb[...], 0.0)
        q_ref[...] = f2 @ a2[...]

    rep = lambda shape: pl.BlockSpec(shape, lambda i: (0, 0))
    return pl.pallas_call(
        body,
        grid=(n // blk,),
        in_specs=[
            pl.BlockSpec((blk, d), lambda i: (i, 0)),
            rep((d, h_dim)), rep((1, h_dim)),
            rep((d, h_dim)), rep((1, h_dim)),
            rep((d, h_dim)), rep((1, h_dim)),
            rep((h_dim, 1)), rep((h_dim, 1)), rep((1, 1)),
        ],
        out_specs=[
            pl.BlockSpec((blk, h_dim), lambda i: (i, 0)),
            pl.BlockSpec((blk, 1), lambda i: (i, 0)),
            pl.BlockSpec((blk, 1), lambda i: (i, 0)),
        ],
        out_shape=[
            jax.ShapeDtypeStruct((n, h_dim), jnp.float32),
            jax.ShapeDtypeStruct((n, 1), jnp.float32),
            jax.ShapeDtypeStruct((n, 1), jnp.float32),
        ],
    )(x, gcn_W, gcn_b.reshape(1, -1), nb_W, nb_b.reshape(1, -1),
      self_W, self_b.reshape(1, -1), attW1, attW2, att_b.reshape(1, 1))


def _dinv_l0(rs_all, l0p, num_edges):
    k, n = rs_all.shape

    def body(rs_ref, l0_ref, dinv_ref, l0o_ref):
        rs = jnp.sum(rs_ref[...], axis=0, keepdims=True) + 1e-10
        dinv_ref[...] = jnp.clip(jax.lax.rsqrt(rs), 0.0, 10.0)
        l0o_ref[...] = (jnp.sum(l0_ref[...], keepdims=True).reshape(1, 1)
                        * (1.0 / num_edges))

    return pl.pallas_call(
        body,
        out_shape=[
            jax.ShapeDtypeStruct((1, n), jnp.float32),
            jax.ShapeDtypeStruct((1, 1), jnp.float32),
        ],
    )(rs_all, l0p)


def _post(hg_raw, hd_raw, m1W, m1b, m2W, m2b, s1W, s1b, s2W, s2b,
          f1W, f1b, f2W, hW, hb):
    n, h_dim = hg_raw.shape
    z_dim = m1W.shape[1]
    c_dim = hW.shape[1]
    blk = 2000 if n % 2000 == 0 else n

    def body(hg_ref, hd_ref, m1, m1b_, m2, m2b_, s1, s1b_, s2, s2b_,
             f1, f1b_, f2, hw, hb_,
             qf_ref, qg_ref, qd_ref, zf_ref, zg_ref, st_ref, w_ref):
        hg = jnp.maximum(hg_ref[...], 0.0)
        hd = jnp.maximum(hd_ref[...], 0.0)
        zg = jnp.maximum(hg @ m1[...] + m1b_[...], 0.0) @ m2[...] + m2b_[...]
        zd = jnp.maximum(hd @ m1[...] + m1b_[...], 0.0) @ m2[...] + m2b_[...]
        st = jax.nn.softplus(
            jnp.maximum(hg @ s1[...] + s1b_[...], 0.0) @ s2[...] + s2b_[...])
        tg = jnp.tanh(zg @ f1[...] + f1b_[...]) @ f2[...]
        td = jnp.tanh(zd @ f1[...] + f1b_[...]) @ f2[...]
        sc = jnp.concatenate([tg, td], axis=1) * 2.0
        w = jax.nn.softmax(sc, axis=1)
        zf = w[:, 0:1] * zg + w[:, 1:2] * zd
        qf_ref[...] = jax.nn.softmax(zf @ hw[...] + hb_[...], axis=1)
        qg_ref[...] = jax.nn.softmax(zg @ hw[...] + hb_[...], axis=1)
        qd_ref[...] = jax.nn.softmax(zd @ hw[...] + hb_[...], axis=1)
        zf_ref[...] = zf
        zg_ref[...] = zg
        st_ref[...] = st
        w_ref[...] = w

    rep = lambda shape: pl.BlockSpec(shape, lambda i: (0, 0))
    blkspec = lambda m: pl.BlockSpec((blk, m), lambda i: (i, 0))
    return pl.pallas_call(
        body,
        grid=(n // blk,),
        in_specs=[
            blkspec(h_dim), blkspec(h_dim),
            rep((h_dim, z_dim)), rep((1, z_dim)),
            rep((z_dim, z_dim)), rep((1, z_dim)),
            rep((h_dim, z_dim)), rep((1, z_dim)),
            rep((z_dim, z_dim)), rep((1, z_dim)),
            rep((z_dim, 32)), rep((1, 32)), rep((32, 1)),
            rep((z_dim, c_dim)), rep((1, c_dim)),
        ],
        out_specs=[
            blkspec(c_dim), blkspec(c_dim), blkspec(c_dim),
            blkspec(z_dim), blkspec(z_dim), blkspec(z_dim), blkspec(2),
        ],
        out_shape=[
            jax.ShapeDtypeStruct((n, c_dim), jnp.float32),
            jax.ShapeDtypeStruct((n, c_dim), jnp.float32),
            jax.ShapeDtypeStruct((n, c_dim), jnp.float32),
            jax.ShapeDtypeStruct((n, z_dim), jnp.float32),
            jax.ShapeDtypeStruct((n, z_dim), jnp.float32),
            jax.ShapeDtypeStruct((n, z_dim), jnp.float32),
            jax.ShapeDtypeStruct((n, 2), jnp.float32),
        ],
    )(hg_raw, hd_raw, m1W, m1b.reshape(1, -1), m2W, m2b.reshape(1, -1),
      s1W, s1b.reshape(1, -1), s2W, s2b.reshape(1, -1),
      f1W, f1b.reshape(1, -1), f2W, hW, hb.reshape(1, -1))


def _logits(z):
    n, zd = z.shape
    blk = 400 if n % 400 == 0 else n

    def body(zi_ref, zj_ref, out_ref):
        out_ref[...] = jax.lax.dot_general(
            zi_ref[...], zj_ref[...], (((1,), (1,)), ((), ())),
            preferred_element_type=jnp.float32)

    return pl.pallas_call(
        body,
        grid=(n // blk,),
        in_specs=[
            pl.BlockSpec((blk, zd), lambda i: (i, 0)),
            pl.BlockSpec((n, zd), lambda i: (0, 0)),
        ],
        out_specs=pl.BlockSpec((blk, n), lambda i: (i, 0)),
        out_shape=jax.ShapeDtypeStruct((n, n), jnp.float32),
    )(z, z)


def _edge_stage_jax(p, q, row, col, vals):
    w = p[row] + q[col]
    sig = jax.nn.sigmoid(w)
    mask = jnp.clip(sig * (_ZETA - _GAMMA) + _GAMMA, 0.0, 1.0)
    mv = vals * mask
    n = p.shape[0]
    rowsum = jax.ops.segment_sum(mv, row, num_segments=n)
    l0p = jnp.sum(jax.nn.sigmoid(w - jnp.log(jnp.float32(-_GAMMA / _ZETA))))
    return mv, rowsum, l0p


def _spmm_jax(vals, h, row, col):
    n = h.shape[0]
    return jax.ops.segment_sum(vals[:, None] * jnp.take(h, col, axis=0),
                               row, num_segments=n)


def kernel(x, adj_indices, adj_values, gcn_W, gcn_b, mean1_W, mean1_b,
           mean2_W, mean2_b, std1_W, std1_b, std2_W, std2_b, nb_W, nb_b,
           self_W, self_b, att_W, att_b, fus1_W, fus1_b, fus2_W,
           head_W, head_b):
    n = x.shape[0]
    e = adj_values.shape[0]
    h_dim = gcn_W.shape[1]
    row = adj_indices[0]
    col = adj_indices[1]

    attW1 = att_W[:h_dim]
    attW2 = att_W[h_dim:]

    h, p2, q2 = _node_pre(x, gcn_W, gcn_b, nb_W, nb_b, self_W, self_b,
                          attW1, attW2, att_b)
    p = p2[:, 0]
    q = q2[:, 0]

    mv, rowsum, l0p = _edge_stage_jax(p, q, row, col, adj_values)
    dinv2, l0o = _dinv_l0(rowsum.reshape(1, n), l0p.reshape(1, 1), float(e))
    d_inv = dinv2[0]
    norm_vals = mv * d_inv[row] * d_inv[col]

    hg_raw = _spmm_jax(adj_values, h, row, col)
    hd_raw = _spmm_jax(norm_vals, h, row, col)

    qf, qg, qd, zf, zg, st, wts = _post(
        hg_raw, hd_raw, mean1_W, mean1_b, mean2_W, mean2_b,
        std1_W, std1_b, std2_W, std2_b, fus1_W, fus1_b, fus2_W,
        head_W, head_b)

    adj_logits = _logits(zg)
    l0_loss = l0o[0, 0]
    return (qf, qg, qd, zf, adj_logits, zg, st, l0_loss,
            wts.reshape(n, 2, 1))


# speedup vs baseline: 3.6956x; 3.6956x over previous
import functools
import math

import jax
import jax.numpy as jnp
from jax.experimental import pallas as pl
from jax.experimental.pallas import tpu as pltpu
from jax.experimental.pallas import tpu_sc as plsc

_GAMMA = -0.5
_ZETA = 1.1


def _node_pre(x, gcn_W, gcn_b, nb_W, nb_b, self_W, self_b, attW1, attW2, att_b):
    n, d = x.shape
    h_dim = gcn_W.shape[1]
    blk = 2000 if n % 2000 == 0 else n

    def body(x_ref, gw, gb, nw, nbb, sw, sb, a1, a2, ab, h_ref, p_ref, q_ref):
        xb = x_ref[...]
        h_ref[...] = xb @ gw[...] + gb[...]
        f1 = jnp.maximum(xb @ nw[...] + nbb[...], 0.0)
        p_ref[...] = f1 @ a1[...] + ab[...]
        f2 = jnp.maximum(xb @ sw[...] + sb[...], 0.0)
        q_ref[...] = f2 @ a2[...]

    rep = lambda shape: pl.BlockSpec(shape, lambda i: (0, 0))
    return pl.pallas_call(
        body,
        grid=(n // blk,),
        in_specs=[
            pl.BlockSpec((blk, d), lambda i: (i, 0)),
            rep((d, h_dim)), rep((1, h_dim)),
            rep((d, h_dim)), rep((1, h_dim)),
            rep((d, h_dim)), rep((1, h_dim)),
            rep((h_dim, 1)), rep((h_dim, 1)), rep((1, 1)),
        ],
        out_specs=[
            pl.BlockSpec((blk, h_dim), lambda i: (i, 0)),
            pl.BlockSpec((blk, 1), lambda i: (i, 0)),
            pl.BlockSpec((blk, 1), lambda i: (i, 0)),
        ],
        out_shape=[
            jax.ShapeDtypeStruct((n, h_dim), jnp.float32),
            jax.ShapeDtypeStruct((n, 1), jnp.float32),
            jax.ShapeDtypeStruct((n, 1), jnp.float32),
        ],
    )(x, gcn_W, gcn_b.reshape(1, -1), nb_W, nb_b.reshape(1, -1),
      self_W, self_b.reshape(1, -1), attW1, attW2, att_b.reshape(1, 1))


def _dinv_l0(rs_all, l0p, num_edges):
    k, n = rs_all.shape

    def body(rs_ref, l0_ref, dinv_ref, l0o_ref):
        rs = jnp.sum(rs_ref[...], axis=0, keepdims=True) + 1e-10
        dinv_ref[...] = jnp.clip(jax.lax.rsqrt(rs), 0.0, 10.0)
        l0o_ref[...] = (jnp.sum(l0_ref[...], keepdims=True).reshape(1, 1)
                        * (1.0 / num_edges))

    return pl.pallas_call(
        body,
        out_shape=[
            jax.ShapeDtypeStruct((1, n), jnp.float32),
            jax.ShapeDtypeStruct((1, 1), jnp.float32),
        ],
    )(rs_all, l0p)


def _post(hg_raw, hd_raw, m1W, m1b, m2W, m2b, s1W, s1b, s2W, s2b,
          f1W, f1b, f2W, hW, hb):
    n, h_dim = hg_raw.shape
    z_dim = m1W.shape[1]
    c_dim = hW.shape[1]
    blk = 2000 if n % 2000 == 0 else n

    def body(hg_ref, hd_ref, m1, m1b_, m2, m2b_, s1, s1b_, s2, s2b_,
             f1, f1b_, f2, hw, hb_,
             qf_ref, qg_ref, qd_ref, zf_ref, zg_ref, st_ref, w_ref):
        hg = jnp.maximum(hg_ref[...], 0.0)
        hd = jnp.maximum(hd_ref[...], 0.0)
        zg = jnp.maximum(hg @ m1[...] + m1b_[...], 0.0) @ m2[...] + m2b_[...]
        zd = jnp.maximum(hd @ m1[...] + m1b_[...], 0.0) @ m2[...] + m2b_[...]
        st = jax.nn.softplus(
            jnp.maximum(hg @ s1[...] + s1b_[...], 0.0) @ s2[...] + s2b_[...])
        tg = jnp.tanh(zg @ f1[...] + f1b_[...]) @ f2[...]
        td = jnp.tanh(zd @ f1[...] + f1b_[...]) @ f2[...]
        sc = jnp.concatenate([tg, td], axis=1) * 2.0
        w = jax.nn.softmax(sc, axis=1)
        zf = w[:, 0:1] * zg + w[:, 1:2] * zd
        qf_ref[...] = jax.nn.softmax(zf @ hw[...] + hb_[...], axis=1)
        qg_ref[...] = jax.nn.softmax(zg @ hw[...] + hb_[...], axis=1)
        qd_ref[...] = jax.nn.softmax(zd @ hw[...] + hb_[...], axis=1)
        zf_ref[...] = zf
        zg_ref[...] = zg
        st_ref[...] = st
        w_ref[...] = w

    rep = lambda shape: pl.BlockSpec(shape, lambda i: (0, 0))
    blkspec = lambda m: pl.BlockSpec((blk, m), lambda i: (i, 0))
    return pl.pallas_call(
        body,
        grid=(n // blk,),
        in_specs=[
            blkspec(h_dim), blkspec(h_dim),
            rep((h_dim, z_dim)), rep((1, z_dim)),
            rep((z_dim, z_dim)), rep((1, z_dim)),
            rep((h_dim, z_dim)), rep((1, z_dim)),
            rep((z_dim, z_dim)), rep((1, z_dim)),
            rep((z_dim, 32)), rep((1, 32)), rep((32, 1)),
            rep((z_dim, c_dim)), rep((1, c_dim)),
        ],
        out_specs=[
            blkspec(c_dim), blkspec(c_dim), blkspec(c_dim),
            blkspec(z_dim), blkspec(z_dim), blkspec(z_dim), blkspec(2),
        ],
        out_shape=[
            jax.ShapeDtypeStruct((n, c_dim), jnp.float32),
            jax.ShapeDtypeStruct((n, c_dim), jnp.float32),
            jax.ShapeDtypeStruct((n, c_dim), jnp.float32),
            jax.ShapeDtypeStruct((n, z_dim), jnp.float32),
            jax.ShapeDtypeStruct((n, z_dim), jnp.float32),
            jax.ShapeDtypeStruct((n, z_dim), jnp.float32),
            jax.ShapeDtypeStruct((n, 2), jnp.float32),
        ],
    )(hg_raw, hd_raw, m1W, m1b.reshape(1, -1), m2W, m2b.reshape(1, -1),
      s1W, s1b.reshape(1, -1), s2W, s2b.reshape(1, -1),
      f1W, f1b.reshape(1, -1), f2W, hW, hb.reshape(1, -1))


def _logits(z):
    n, zd = z.shape
    blk = 400 if n % 400 == 0 else n

    def body(zi_ref, zj_ref, out_ref):
        out_ref[...] = jax.lax.dot_general(
            zi_ref[...], zj_ref[...], (((1,), (1,)), ((), ())),
            preferred_element_type=jnp.float32)

    return pl.pallas_call(
        body,
        grid=(n // blk,),
        in_specs=[
            pl.BlockSpec((blk, zd), lambda i: (i, 0)),
            pl.BlockSpec((n, zd), lambda i: (0, 0)),
        ],
        out_specs=pl.BlockSpec((blk, n), lambda i: (i, 0)),
        out_shape=jax.ShapeDtypeStruct((n, n), jnp.float32),
    )(z, z)


def _edge_stage_sc(row2, col2, vals2, p, q, e_real):
    cr = row2.shape[0]
    n = p.shape[0]
    nw = 32
    cpt = cr // nw
    shift = float(-math.log(-_GAMMA / _ZETA))
    scale = _ZETA - _GAMMA
    mesh = plsc.VectorSubcoreMesh(core_axis_name="c", subcore_axis_name="s")

    @functools.partial(
        pl.kernel,
        out_type=[
            jax.ShapeDtypeStruct((cr, 128), jnp.float32),
            jax.ShapeDtypeStruct((nw, 1, n), jnp.float32),
            jax.ShapeDtypeStruct((nw, 1, 16), jnp.float32),
        ],
        mesh=mesh,
        scratch_types=[
            pltpu.VMEM((cpt, 128), jnp.int32),
            pltpu.VMEM((cpt, 128), jnp.int32),
            pltpu.VMEM((cpt, 128), jnp.float32),
            pltpu.VMEM((cpt, 128), jnp.float32),
            pltpu.VMEM((n,), jnp.float32),
            pltpu.VMEM((n,), jnp.float32),
            pltpu.VMEM((1, n), jnp.float32),
            pltpu.VMEM((1, 16), jnp.float32),
        ],
        compiler_params=pltpu.CompilerParams(needs_layout_passes=False),
    )
    def k(row_h, col_h, vals_h, p_h, q_h, mv_h, rs_h, l0_h,
          row_v, col_v, vals_v, mv_v, p_v, q_v, rs_v, l0_v):
        wid = jax.lax.axis_index("s") * 2 + jax.lax.axis_index("c")
        base = wid * cpt
        pltpu.sync_copy(row_h.at[pl.ds(base, cpt)], row_v)
        pltpu.sync_copy(col_h.at[pl.ds(base, cpt)], col_v)
        pltpu.sync_copy(vals_h.at[pl.ds(base, cpt)], vals_v)
        pltpu.sync_copy(p_h, p_v)
        pltpu.sync_copy(q_h, q_v)

        zero = jnp.zeros((16,), jnp.float32)
        zi16 = jnp.zeros((16,), jnp.int32)

        def zb(i, carry):
            rs_v[0, pl.ds(i * 16, 16)] = zero
            return carry

        jax.lax.fori_loop(0, n // 16, zb, 0)

        def eb(i, l0acc):
            def sub(j, acc2):
                sl = pl.ds(j * 16, 16)
                r16 = row_v[i, sl]
                c16 = col_v[i, sl]
                v16 = vals_v[i, sl]
                pg = plsc.load_gather(p_v, [r16])
                qg = plsc.load_gather(q_v, [c16])
                w = pg + qg
                sig = 1.0 / (1.0 + jnp.exp(-w))
                m = jnp.clip(sig * scale + _GAMMA, 0.0, 1.0)
                mv16 = v16 * m
                gidx = ((base + i) * 128 + j * 16
                        + jax.lax.iota(jnp.int32, 16))
                valid = gidx < e_real
                mv16 = jnp.where(valid, mv16, 0.0)
                mv_v[i, sl] = mv16
                plsc.addupdate_scatter(rs_v, [zi16, r16], mv16)
                l0t = 1.0 / (1.0 + jnp.exp(-(w + shift)))
                return acc2 + jnp.where(valid, l0t, 0.0)

            return jax.lax.fori_loop(0, 8, sub, l0acc)

        l0acc = jax.lax.fori_loop(0, cpt, eb, jnp.zeros((16,), jnp.float32))
        l0_v[0, pl.ds(0, 16)] = l0acc
        pltpu.sync_copy(mv_v, mv_h.at[pl.ds(base, cpt)])
        pltpu.sync_copy(rs_v, rs_h.at[wid])
        pltpu.sync_copy(l0_v, l0_h.at[wid])

    return k(row2, col2, vals2, p, q)


def _spmm_both_sc(row16, col16, vals2, mv2, dinv, h):
    cr16 = row16.shape[0]
    n, hd = h.shape
    ns = 16
    cr = vals2.shape[0]
    cpt = cr // ns
    blk = min(8, cpt)
    nblk = cpt // blk
    npad = ((n + 127) // 128) * 128
    rpt = npad // ns
    zrows = 8
    assert rpt % zrows == 0
    mesh = plsc.VectorSubcoreMesh(core_axis_name="c", subcore_axis_name="s")

    @functools.partial(
        pl.kernel,
        out_type=jax.ShapeDtypeStruct((2 * npad, hd), jnp.float32),
        mesh=mesh,
        scratch_types=[
            pltpu.VMEM((blk * 8, 17), jnp.int32),
            pltpu.VMEM((blk * 8, 17), jnp.int32),
            pltpu.VMEM((blk, 128), jnp.float32),
            pltpu.VMEM((blk, 128), jnp.float32),
            pltpu.VMEM((n,), jnp.float32),
            pltpu.VMEM((16,), jnp.float32),
            pltpu.VMEM((17, hd), jnp.float32),
            pltpu.VMEM((zrows, hd), jnp.float32),
            pltpu.VMEM_SHARED((npad, hd), jnp.float32),
            pltpu.SemaphoreType.DMA,
        ],
        compiler_params=pltpu.CompilerParams(needs_layout_passes=False),
    )
    def k(row_h, col_h, vg_h, mv_h, dinv_h, h_h, out_h,
          ridx_v, cidx_v, vg_v, mv_v, dinv_v, val_s, rows_v, z_v, acc, sem):
        c = jax.lax.axis_index("c")
        s = jax.lax.axis_index("s")
        base = s * cpt
        pltpu.sync_copy(dinv_h, dinv_v)
        is_gen = c == 0

        zero = jnp.zeros((16,), jnp.float32)

        def zb(i, carry):
            def zb2(j, carry2):
                z_v[i, pl.ds(j * 16, 16)] = zero
                return carry2
            return jax.lax.fori_loop(0, hd // 16, zb2, carry)

        jax.lax.fori_loop(0, zrows, zb, 0)

        def zc(b, carry):
            pltpu.sync_copy(z_v, acc.at[pl.ds(s * rpt + b * zrows, zrows)])
            return carry

        jax.lax.fori_loop(0, rpt // zrows, zc, 0)
        plsc.subcore_barrier()

        lane_ids = [jnp.full((16,), j2, jnp.int32) for j2 in range(16)]

        def bb(b, carry):
            roff = base + b * blk
            pltpu.sync_copy(row_h.at[pl.ds(roff * 8, blk * 8)], ridx_v)
            pltpu.sync_copy(col_h.at[pl.ds(roff * 8, blk * 8)], cidx_v)
            pltpu.sync_copy(vg_h.at[pl.ds(roff, blk)], vg_v)
            pltpu.sync_copy(mv_h.at[pl.ds(roff, blk)], mv_v)

            def rb(i, carry2):
                def sub(j, carry3):
                    kk = i * 8 + j
                    sl = pl.ds(j * 16, 16)
                    r16 = ridx_v[kk, pl.ds(1, 16)]
                    c16 = cidx_v[kk, pl.ds(1, 16)]
                    g1 = plsc.load_gather(dinv_v, [r16])
                    g2 = plsc.load_gather(dinv_v, [c16])
                    vd = mv_v[i, sl] * g1 * g2
                    val16 = jnp.where(is_gen, vg_v[i, sl], vd)
                    pltpu.sync_copy(h_h.at[cidx_v.at[kk]], rows_v)
                    for j2 in range(16):
                        vv = jnp.broadcast_to(val16[j2], (16,))
                        for pcol in range(hd // 16):
                            s2 = pl.ds(pcol * 16, 16)
                            rows_v[j2 + 1, s2] = rows_v[j2 + 1, s2] * vv
                    pltpu.sync_copy(rows_v, acc.at[ridx_v.at[kk]], add=True)
                    return carry3

                return jax.lax.fori_loop(0, 8, sub, carry2)

            return jax.lax.fori_loop(0, blk, rb, carry)

        jax.lax.fori_loop(0, nblk, bb, 0)
        plsc.subcore_barrier()
        pltpu.sync_copy(acc.at[pl.ds(s * rpt, rpt)],
                        out_h.at[pl.ds(c * npad + s * rpt, rpt)])

    return k(row16, col16, vals2, mv2, dinv, h)


def kernel(x, adj_indices, adj_values, gcn_W, gcn_b, mean1_W, mean1_b,
           mean2_W, mean2_b, std1_W, std1_b, std2_W, std2_b, nb_W, nb_b,
           self_W, self_b, att_W, att_b, fus1_W, fus1_b, fus2_W,
           head_W, head_b):
    n = x.shape[0]
    e = adj_values.shape[0]
    h_dim = gcn_W.shape[1]
    row = adj_indices[0]
    col = adj_indices[1]

    attW1 = att_W[:h_dim]
    attW2 = att_W[h_dim:]

    h, p2, q2 = _node_pre(x, gcn_W, gcn_b, nb_W, nb_b, self_W, self_b,
                          attW1, attW2, att_b)
    p = p2[:, 0]
    q = q2[:, 0]

    epad = ((e + 4095) // 4096) * 4096
    pad = epad - e
    if pad:
        row_p = jnp.concatenate([row, jnp.zeros((pad,), row.dtype)])
        col_p = jnp.concatenate([col, jnp.zeros((pad,), col.dtype)])
        vals_p = jnp.concatenate([adj_values,
                                  jnp.zeros((pad,), adj_values.dtype)])
    else:
        row_p, col_p, vals_p = row, col, adj_values
    row2 = row_p.reshape(-1, 128)
    col2 = col_p.reshape(-1, 128)
    vals2 = vals_p.reshape(-1, 128)

    mv2, rs_all, l0p = _edge_stage_sc(row2, col2, vals2, p, q, e)
    dinv2, l0o = _dinv_l0(rs_all.reshape(32, n), l0p.reshape(32, 16),
                          float(e))
    cr16 = epad // 16
    row17 = jnp.concatenate(
        [jnp.full((cr16, 1), n, jnp.int32), row_p.reshape(-1, 16)], axis=1)
    col17 = jnp.concatenate(
        [jnp.zeros((cr16, 1), jnp.int32), col_p.reshape(-1, 16)], axis=1)
    hid2 = _spmm_both_sc(row17, col17, vals2, mv2, dinv2[0], h)
    npad = ((n + 127) // 128) * 128
    hg_raw = hid2[:n]
    hd_raw = hid2[npad:npad + n]

    qf, qg, qd, zf, zg, st, wts = _post(
        hg_raw, hd_raw, mean1_W, mean1_b, mean2_W, mean2_b,
        std1_W, std1_b, std2_W, std2_b, fus1_W, fus1_b, fus2_W,
        head_W, head_b)

    adj_logits = _logits(zg)
    l0_loss = l0o[0, 0]
    return (qf, qg, qd, zf, adj_logits, zg, st, l0_loss,
            wts.reshape(n, 2, 1))
